# baseline (speedup 1.0000x reference)
import functools

import jax
import jax.numpy as jnp
from jax import lax
from jax.experimental import pallas as pl
from jax.experimental.pallas import tpu as pltpu
from jax.experimental.pallas import tpu_sc as plsc

N = 4096
E = 131072
D = 128
NC = 2
NS = 16
L = 16
NW = NC * NS
EPW = E // NW
NPAD = 4608
TPR = NPAD // NS
DPW = N // NW
SCK = 512
NST = EPW // SCK

_mesh = plsc.VectorSubcoreMesh(core_axis_name="c", subcore_axis_name="s")


@functools.partial(
    pl.kernel,
    mesh=_mesh,
    out_type=(
        jax.ShapeDtypeStruct((NC * N,), jnp.float32),
        jax.ShapeDtypeStruct((N * N,), jnp.int32),
    ),
    scratch_types=(
        tuple(pltpu.VMEM((SCK,), jnp.int32) for _ in range(2)),
        tuple(pltpu.VMEM((SCK,), jnp.int32) for _ in range(2)),
        tuple(pltpu.VMEM((SCK,), jnp.int32) for _ in range(2)),
        tuple(pltpu.VMEM((SCK,), jnp.int32) for _ in range(2)),
        pltpu.VMEM((SCK,), jnp.float32),
        pltpu.VMEM((256,), jnp.float32),
        pltpu.VMEM_SHARED((N,), jnp.float32),
        pltpu.SemaphoreType.DMA,
        pltpu.SemaphoreType.DMA,
    ),
)
def _p1(row_hbm, col_hbm, deg_out, table_out,
        rowf, colf, keys, eidv, ones, zbuf, sdeg, semL, semA):
    c = lax.axis_index("c")
    s = lax.axis_index("s")
    wid = s * NC + c
    lane = lax.iota(jnp.int32, L)
    for g in range(SCK // L):
        ones[pl.ds(g * L, L)] = jnp.full((L,), 1.0, jnp.float32)
    for g in range(256 // L):
        zbuf[pl.ds(g * L, L)] = jnp.zeros((L,), jnp.float32)
    pltpu.sync_copy(zbuf, sdeg.at[pl.ds(s * 256, 256)])
    plsc.subcore_barrier()

    ebase = wid * EPW

    def fire_lin(i, b):
        base = ebase + i * SCK
        return (pltpu.async_copy(row_hbm.at[pl.ds(base, SCK)], rowf[b], semL),
                pltpu.async_copy(col_hbm.at[pl.ds(base, SCK)], colf[b], semL))

    lin = {0: fire_lin(0, 0)}
    gat = {}
    for i in range(NST):
        b = i & 1
        for h in lin[i]:
            h.wait()
        if i + 1 < NST:
            lin[i + 1] = fire_lin(i + 1, b ^ 1)
        if i >= 2:
            gat[i - 2].wait()
        base = ebase + i * SCK
        for g in range(SCK // L):
            off = g * L
            r = rowf[b][pl.ds(off, L)]
            cc = colf[b][pl.ds(off, L)]
            keys[b][pl.ds(off, L)] = (r << 12) | cc
            eidv[b][pl.ds(off, L)] = (base + off) + lane
        gat[i] = pltpu.async_copy(eidv[b], table_out.at[keys[b]], semA)
        pltpu.sync_copy(ones, sdeg.at[colf[b]], add=True)
    for i in (NST - 2, NST - 1):
        gat[i].wait()
    plsc.subcore_barrier()
    pltpu.sync_copy(sdeg.at[pl.ds(s * 256, 256)], zbuf)
    pltpu.sync_copy(zbuf, deg_out.at[pl.ds(c * N + s * 256, 256)])


def _p1b_body(degp_ref, x_ref, y_ref):
    deg = degp_ref[0] + degp_ref[1]
    ok = deg > 0.0
    dis = jnp.where(ok, lax.rsqrt(jnp.where(ok, deg, 1.0)), 0.0)
    y_ref[...] = dis * x_ref[...]


def _p1b(degp, x):
    return pl.pallas_call(
        _p1b_body,
        grid=(N // 128,),
        in_specs=[
            pl.BlockSpec((NC, 128, 1), lambda i: (0, i, 0)),
            pl.BlockSpec((128, D), lambda i: (i, 0)),
        ],
        out_specs=pl.BlockSpec((128, D), lambda i: (i, 0)),
        out_shape=jax.ShapeDtypeStruct((N, D), jnp.float32),
    )(degp, x)


@functools.partial(
    pl.kernel,
    mesh=_mesh,
    out_type=(
        jax.ShapeDtypeStruct((NC, NPAD, D), jnp.float32),
        jax.ShapeDtypeStruct((N,), jnp.float32),
    ),
    scratch_types=(
        tuple(pltpu.VMEM((SCK,), jnp.int32) for _ in range(2)),
        tuple(pltpu.VMEM((SCK,), jnp.int32) for _ in range(2)),
        pltpu.VMEM((SCK,), jnp.int32),
        pltpu.VMEM((SCK,), jnp.int32),
        pltpu.VMEM((SCK,), jnp.int32),
        pltpu.VMEM((SCK, D), jnp.float32),
        pltpu.VMEM((DPW,), jnp.int32),
        pltpu.VMEM((DPW,), jnp.int32),
        pltpu.VMEM((DPW,), jnp.int32),
        pltpu.VMEM((DPW,), jnp.int32),
        pltpu.VMEM((DPW,), jnp.float32),
        pltpu.VMEM((8, D), jnp.float32),
        pltpu.VMEM((96, D), jnp.float32),
        pltpu.VMEM_SHARED((NPAD, D), jnp.float32),
        pltpu.SemaphoreType.DMA,
        pltpu.SemaphoreType.DMA,
        pltpu.SemaphoreType.DMA,
    ),
)
def _p2(row_hbm, col_hbm, table_hbm, y_hbm, acc_out, self_out,
        rowf, colf, keyf, tidf, row2s, rows, dkey, dtid, drow, dcol, dval,
        zrows, obuf, sacc, semL, semA, semB):
    c = lax.axis_index("c")
    s = lax.axis_index("s")
    wid = s * NC + c
    lane = lax.iota(jnp.int32, L)
    for r in range(8):
        for g in range(D // L):
            zrows[r, pl.ds(g * L, L)] = jnp.zeros((L,), jnp.float32)
    for k in range(TPR // 8):
        pltpu.sync_copy(zrows, sacc.at[pl.ds(s * TPR + k * 8, 8)])
    plsc.subcore_barrier()

    ebase = wid * EPW
    junk = 4096 + s * 16

    def fire_lin(i, b):
        base = ebase + i * SCK
        return (pltpu.async_copy(row_hbm.at[pl.ds(base, SCK)], rowf[b], semL),
                pltpu.async_copy(col_hbm.at[pl.ds(base, SCK)], colf[b], semL))

    lin = {0: fire_lin(0, 0)}
    for i in range(NST):
        b = i & 1
        for h in lin[i]:
            h.wait()
        if i + 1 < NST:
            lin[i + 1] = fire_lin(i + 1, b ^ 1)
        hy = pltpu.async_copy(y_hbm.at[colf[b]], rows, semB)
        base = ebase + i * SCK
        for g in range(SCK // L):
            off = g * L
            r = rowf[b][pl.ds(off, L)]
            cc = colf[b][pl.ds(off, L)]
            keyf[pl.ds(off, L)] = (r << 12) | cc
        ht = pltpu.async_copy(table_hbm.at[keyf], tidf, semA)
        ht.wait()
        for g in range(SCK // L):
            off = g * L
            tid = tidf[pl.ds(off, L)]
            eid = (base + off) + lane
            canon = tid == eid
            r = rowf[b][pl.ds(off, L)]
            row2s[pl.ds(off, L)] = jnp.where(canon, r, junk)
        hy.wait()
        pltpu.sync_copy(rows, sacc.at[row2s], add=True)

    dbase = wid * DPW
    for g in range(DPW // L):
        idx = (dbase + g * L) + lane
        dkey[pl.ds(g * L, L)] = idx * 4097
    pltpu.sync_copy(table_hbm.at[dkey], dtid)
    for g in range(DPW // L):
        tid = dtid[pl.ds(g * L, L)]
        dkey[pl.ds(g * L, L)] = jnp.clip(tid, 0, E - 1)
    ha = pltpu.async_copy(row_hbm.at[dkey], drow, semA)
    hb = pltpu.async_copy(col_hbm.at[dkey], dcol, semB)
    ha.wait()
    hb.wait()
    for g in range(DPW // L):
        idx = (dbase + g * L) + lane
        hit = (drow[pl.ds(g * L, L)] == idx) & (dcol[pl.ds(g * L, L)] == idx)
        dval[pl.ds(g * L, L)] = jnp.where(hit, 1.0, 0.0)
    pltpu.sync_copy(dval, self_out.at[pl.ds(dbase, DPW)])

    plsc.subcore_barrier()
    for k in range(TPR // 96):
        pltpu.sync_copy(sacc.at[pl.ds(s * TPR + k * 96, 96)], obuf)
        pltpu.sync_copy(obuf, acc_out.at[c, pl.ds(s * TPR + k * 96, 96)])


def _p3_body(x_ref, degp_ref, acc_ref, self_ref, w_ref, z1_ref, z2_ref):
    deg = degp_ref[0] + degp_ref[1]
    ok = deg > 0.0
    dis = jnp.where(ok, lax.rsqrt(jnp.where(ok, deg, 1.0)), 0.0)
    a = acc_ref[0] + acc_ref[1]
    sm = self_ref[...]
    xb = x_ref[...]
    h2 = dis * a
    h1 = xb - h2 + (sm * dis * dis) * xb
    w = w_ref[...]
    z1_ref[...] = jnp.maximum(
        jnp.dot(h1, w, preferred_element_type=jnp.float32), 0.0)
    z2_ref[...] = jnp.maximum(
        jnp.dot(h2, w, preferred_element_type=jnp.float32), 0.0)


def _p3(x, degp, acc, selfp, W):
    return pl.pallas_call(
        _p3_body,
        grid=(N // 128,),
        in_specs=[
            pl.BlockSpec((128, D), lambda i: (i, 0)),
            pl.BlockSpec((NC, 128, 1), lambda i: (0, i, 0)),
            pl.BlockSpec((NC, 128, D), lambda i: (0, i, 0)),
            pl.BlockSpec((128, 1), lambda i: (i, 0)),
            pl.BlockSpec((D, D), lambda i: (0, 0)),
        ],
        out_specs=[
            pl.BlockSpec((128, D), lambda i: (i, 0)),
            pl.BlockSpec((128, D), lambda i: (i, 0)),
        ],
        out_shape=[
            jax.ShapeDtypeStruct((N, D), jnp.float32),
            jax.ShapeDtypeStruct((N, D), jnp.float32),
        ],
    )(x, degp, acc, selfp, W)


def kernel(x, edge_index, W):
    row = edge_index[0]
    col = edge_index[1]
    deg_part, table = _p1(row, col)
    degp = deg_part.reshape(NC, N, 1)
    y = _p1b(degp, x)
    acc, selfv = _p2(row, col, table, y)
    selfp = selfv.reshape(N, 1)
    z1, z2 = _p3(x, degp, acc, selfp, W)
    return (z2, z1, z2)


# speedup vs baseline: 1.0573x; 1.0573x over previous
import functools

import jax
import jax.numpy as jnp
from jax import lax
from jax.experimental import pallas as pl
from jax.experimental.pallas import tpu as pltpu
from jax.experimental.pallas import tpu_sc as plsc

N = 4096
E = 131072
D = 128
NC = 2
NS = 16
L = 16
NW = NC * NS
EPW = E // NW
SCK = 512
NSC = EPW // SCK
NJ = SCK // 128
NPAD = 4608
TPR = NPAD // NS

_mesh = plsc.VectorSubcoreMesh(core_axis_name="c", subcore_axis_name="s")


@functools.partial(
    pl.kernel,
    mesh=_mesh,
    out_type=(
        jax.ShapeDtypeStruct((NC * N,), jnp.float32),
        jax.ShapeDtypeStruct((N * N,), jnp.int32),
    ),
    scratch_types=(
        pltpu.VMEM((SCK,), jnp.int32),
        tuple(pltpu.VMEM((128,), jnp.int32) for _ in range(NJ)),
        tuple(pltpu.VMEM((128,), jnp.int32) for _ in range(NJ)),
        pltpu.VMEM((NJ, 128), jnp.int32),
        pltpu.VMEM((128,), jnp.float32),
        pltpu.VMEM((256,), jnp.float32),
        pltpu.VMEM_SHARED((N,), jnp.float32),
        pltpu.SemaphoreType.DMA,
    ),
)
def _p1(row_hbm, col_hbm, deg_out, table_out,
        row_flat, cols, keys, eidv, ones, zbuf, sdeg, sem):
    c = lax.axis_index("c")
    s = lax.axis_index("s")
    wid = s * NC + c
    lane = lax.iota(jnp.int32, L)
    for g in range(128 // L):
        ones[pl.ds(g * L, L)] = jnp.full((L,), 1.0, jnp.float32)
    for g in range(256 // L):
        zbuf[pl.ds(g * L, L)] = jnp.zeros((L,), jnp.float32)
    pltpu.sync_copy(zbuf, sdeg.at[pl.ds(s * 256, 256)])
    plsc.subcore_barrier()

    ebase = wid * EPW

    def body(i, carry):
        base = ebase + i * SCK
        pltpu.sync_copy(row_hbm.at[pl.ds(base, SCK)], row_flat)
        for j in range(NJ):
            pltpu.sync_copy(col_hbm.at[pl.ds(base + j * 128, 128)], cols[j])
        for j in range(NJ):
            for g in range(128 // L):
                off = j * 128 + g * L
                r = row_flat[pl.ds(off, L)]
                cc = cols[j][pl.ds(g * L, L)]
                keys[j][pl.ds(g * L, L)] = (r << 12) | cc
                eidv[j, pl.ds(g * L, L)] = (base + off) + lane
        cps = []
        for j in range(NJ):
            cps.append(pltpu.async_copy(eidv.at[j], table_out.at[keys[j]], sem))
        for j in range(NJ):
            pltpu.sync_copy(ones, sdeg.at[cols[j]], add=True)
        for cp in cps:
            cp.wait()
        return carry

    lax.fori_loop(0, NSC, body, 0)
    plsc.subcore_barrier()
    pltpu.sync_copy(sdeg.at[pl.ds(s * 256, 256)], zbuf)
    pltpu.sync_copy(zbuf, deg_out.at[pl.ds(c * N + s * 256, 256)])


def _p1b_body(degp_ref, x_ref, y_ref):
    deg = degp_ref[0] + degp_ref[1]
    ok = deg > 0.0
    dis = jnp.where(ok, lax.rsqrt(jnp.where(ok, deg, 1.0)), 0.0)
    y_ref[...] = dis * x_ref[...]


def _p1b(degp, x):
    return pl.pallas_call(
        _p1b_body,
        grid=(N // 128,),
        in_specs=[
            pl.BlockSpec((NC, 128, 1), lambda i: (0, i, 0)),
            pl.BlockSpec((128, D), lambda i: (i, 0)),
        ],
        out_specs=pl.BlockSpec((128, D), lambda i: (i, 0)),
        out_shape=jax.ShapeDtypeStruct((N, D), jnp.float32),
    )(degp, x)


@functools.partial(
    pl.kernel,
    mesh=_mesh,
    out_type=(
        jax.ShapeDtypeStruct((NC, NPAD, D), jnp.float32),
        jax.ShapeDtypeStruct((NC * NPAD,), jnp.float32),
    ),
    scratch_types=(
        pltpu.VMEM((SCK,), jnp.int32),
        pltpu.VMEM((SCK,), jnp.int32),
        pltpu.VMEM((SCK,), jnp.int32),
        pltpu.VMEM((SCK,), jnp.int32),
        tuple(pltpu.VMEM((128,), jnp.int32) for _ in range(NJ)),
        pltpu.VMEM((NJ, 128), jnp.float32),
        pltpu.VMEM((NJ, 128, D), jnp.float32),
        pltpu.VMEM((8, D), jnp.float32),
        pltpu.VMEM((TPR,), jnp.float32),
        pltpu.VMEM((96, D), jnp.float32),
        pltpu.VMEM_SHARED((NPAD, D), jnp.float32),
        pltpu.VMEM_SHARED((NPAD,), jnp.float32),
        pltpu.SemaphoreType.DMA,
        pltpu.SemaphoreType.DMA,
        pltpu.SemaphoreType.DMA,
    ),
)
def _p2(row_hbm, col_hbm, table_hbm, y_hbm, acc_out, self_out,
        row_flat, col_flat, key_flat, tid_flat, row2s, sval, rows4,
        zrows, zvec, obuf, sacc, sself, semA, semB, semC):
    c = lax.axis_index("c")
    s = lax.axis_index("s")
    wid = s * NC + c
    lane = lax.iota(jnp.int32, L)
    for r in range(8):
        for g in range(D // L):
            zrows[r, pl.ds(g * L, L)] = jnp.zeros((L,), jnp.float32)
    for g in range(TPR // L):
        zvec[pl.ds(g * L, L)] = jnp.zeros((L,), jnp.float32)
    for k in range(TPR // 8):
        pltpu.sync_copy(zrows, sacc.at[pl.ds(s * TPR + k * 8, 8)])
    pltpu.sync_copy(zvec, sself.at[pl.ds(s * TPR, TPR)])
    plsc.subcore_barrier()

    ebase = wid * EPW
    junk = 4096 + s * 16

    def body(i, carry):
        base = ebase + i * SCK
        pltpu.sync_copy(row_hbm.at[pl.ds(base, SCK)], row_flat)
        pltpu.sync_copy(col_hbm.at[pl.ds(base, SCK)], col_flat)
        for j in range(NJ):
            for g in range(128 // L):
                off = j * 128 + g * L
                r = row_flat[pl.ds(off, L)]
                cc = col_flat[pl.ds(off, L)]
                key_flat[pl.ds(off, L)] = (r << 12) | cc
        gath = []
        for j in range(NJ):
            gath.append(pltpu.async_copy(
                table_hbm.at[key_flat.at[pl.ds(j * 128, 128)]],
                tid_flat.at[pl.ds(j * 128, 128)], semA))
            gath.append(pltpu.async_copy(
                y_hbm.at[col_flat.at[pl.ds(j * 128, 128)]], rows4.at[j], semB))
        for j in range(NJ):
            gath[2 * j].wait()
            for g in range(128 // L):
                off = j * 128 + g * L
                tid = tid_flat[pl.ds(off, L)]
                eid = (base + off) + lane
                canon = tid == eid
                r = row_flat[pl.ds(off, L)]
                cc = col_flat[pl.ds(off, L)]
                row2s[j][pl.ds(g * L, L)] = jnp.where(canon, r, junk)
                sval[j, pl.ds(g * L, L)] = jnp.where(
                    canon & (r == cc), 1.0, 0.0)
        for j in range(NJ):
            gath[2 * j + 1].wait()
            pltpu.sync_copy(rows4.at[j], sacc.at[row2s[j]], add=True)
            pltpu.sync_copy(sval.at[j], sself.at[row2s[j]], add=True)
        return carry

    lax.fori_loop(0, NSC, body, 0)
    plsc.subcore_barrier()
    for k in range(TPR // 96):
        pltpu.sync_copy(sacc.at[pl.ds(s * TPR + k * 96, 96)], obuf)
        pltpu.sync_copy(obuf, acc_out.at[c, pl.ds(s * TPR + k * 96, 96)])
    pltpu.sync_copy(sself.at[pl.ds(s * TPR, TPR)], zvec)
    pltpu.sync_copy(zvec, self_out.at[pl.ds(c * NPAD + s * TPR, TPR)])


def _p3_body(x_ref, degp_ref, acc_ref, self_ref, w_ref, z1_ref, z2_ref):
    deg = degp_ref[0] + degp_ref[1]
    ok = deg > 0.0
    dis = jnp.where(ok, lax.rsqrt(jnp.where(ok, deg, 1.0)), 0.0)
    a = acc_ref[0] + acc_ref[1]
    sm = self_ref[0] + self_ref[1]
    xb = x_ref[...]
    h2 = dis * a
    h1 = xb - h2 + (sm * dis * dis) * xb
    w = w_ref[...]
    z1_ref[...] = jnp.maximum(
        jnp.dot(h1, w, preferred_element_type=jnp.float32), 0.0)
    z2_ref[...] = jnp.maximum(
        jnp.dot(h2, w, preferred_element_type=jnp.float32), 0.0)


def _p3(x, degp, acc, selfp, W):
    return pl.pallas_call(
        _p3_body,
        grid=(N // 128,),
        in_specs=[
            pl.BlockSpec((128, D), lambda i: (i, 0)),
            pl.BlockSpec((NC, 128, 1), lambda i: (0, i, 0)),
            pl.BlockSpec((NC, 128, D), lambda i: (0, i, 0)),
            pl.BlockSpec((NC, 128, 1), lambda i: (0, i, 0)),
            pl.BlockSpec((D, D), lambda i: (0, 0)),
        ],
        out_specs=[
            pl.BlockSpec((128, D), lambda i: (i, 0)),
            pl.BlockSpec((128, D), lambda i: (i, 0)),
        ],
        out_shape=[
            jax.ShapeDtypeStruct((N, D), jnp.float32),
            jax.ShapeDtypeStruct((N, D), jnp.float32),
        ],
    )(x, degp, acc, selfp, W)


def kernel(x, edge_index, W):
    row = edge_index[0]
    col = edge_index[1]
    deg_part, table = _p1(row, col)
    degp = deg_part.reshape(NC, N, 1)
    y = _p1b(degp, x)
    acc, selfv = _p2(row, col, table, y)
    selfp = selfv.reshape(NC, NPAD)[:, :N].reshape(NC, N, 1)
    z1, z2 = _p3(x, degp, acc, selfp, W)
    return (z2, z1, z2)
